# baseline (speedup 1.0000x reference)
import functools

import jax
import jax.numpy as jnp
from jax import lax
from jax.experimental import pallas as pl
from jax.experimental.pallas import tpu as pltpu
from jax.experimental.pallas import tpu_sc as plsc

N = 10000
D = 128
NC = 2
NS = 16
NW = NC * NS
EPW = 10240
BK = 64
K = EPW // BK
E_PAD = NW * EPW
TOTAL_B = E_PAD // BK
CB = 8
K0 = 296
K1 = 2 * K - K0
RPT = 640
N_ACC = NS * RPT


def _sc_segsum_body(h_hbm, srcs_hbm, dsts_hbm, zd_hbm,
                    s_out, src_v, dst_v, rows0_v, rows1_v, acc_sh,
                    sem0, sem1):
    c = lax.axis_index("c")
    s = lax.axis_index("s")
    r0 = s * RPT
    bufs = (rows0_v, rows1_v)
    sems = (sem0, sem1)
    pltpu.sync_copy(zd_hbm, rows0_v)
    for b in range(RPT // BK):
        pltpu.sync_copy(rows0_v, acc_sh.at[pl.ds(r0 + b * BK, BK)])
    plsc.subcore_barrier()

    base = jnp.where(c == 0, s * K0, NS * K0 + s * K1)
    nchunks = jnp.where(c == 0, K0 // CB, K1 // CB)

    def outer(t, carry):
        b0 = base + t * CB
        pltpu.sync_copy(srcs_hbm.at[pl.ds(b0, CB)], src_v)
        pltpu.sync_copy(dsts_hbm.at[pl.ds(b0, CB)], dst_v)
        desc = [None, None]
        desc[0] = pltpu.async_copy(h_hbm.at[src_v.at[0]], bufs[0], sems[0])
        for j in range(CB):
            b = j & 1
            if j + 1 < CB:
                desc[1 - b] = pltpu.async_copy(
                    h_hbm.at[src_v.at[j + 1]], bufs[1 - b], sems[1 - b])
            desc[b].wait()
            pltpu.sync_copy(bufs[b], acc_sh.at[dst_v.at[j]], add=True)
        return carry

    lax.fori_loop(0, nchunks, outer, 0)
    plsc.subcore_barrier()
    for b in range(RPT // BK):
        pltpu.sync_copy(acc_sh.at[pl.ds(r0 + b * BK, BK)], rows0_v)
        pltpu.sync_copy(rows0_v, s_out.at[c, pl.ds(r0 + b * BK, BK)])


def _sc_count_body(dsts_hbm, zd_hbm, ones_hbm,
                   c_out, dst_v, rows_v, ones_v, acc_sh):
    c = lax.axis_index("c")
    s = lax.axis_index("s")
    wid = c * NS + s
    r0 = s * RPT
    pltpu.sync_copy(zd_hbm, rows_v)
    for b in range(RPT // BK):
        pltpu.sync_copy(rows_v, acc_sh.at[pl.ds(r0 + b * BK, BK)])
    pltpu.sync_copy(ones_hbm, ones_v)
    plsc.subcore_barrier()

    def outer(t, carry):
        pltpu.sync_copy(dsts_hbm.at[pl.ds(wid * K + t * CB, CB)], dst_v)

        def step(j, carry2):
            pltpu.sync_copy(ones_v, acc_sh.at[dst_v.at[j]], add=True)
            return carry2

        lax.fori_loop(0, CB, step, 0)
        return carry

    lax.fori_loop(0, K // CB, outer, 0)
    plsc.subcore_barrier()
    for b in range(RPT // BK):
        pltpu.sync_copy(acc_sh.at[pl.ds(r0 + b * BK, BK)], rows_v)
        pltpu.sync_copy(rows_v, c_out.at[c, pl.ds(r0 + b * BK, BK)])


def _sc_mesh():
    return plsc.VectorSubcoreMesh(core_axis_name="c", subcore_axis_name="s")


@functools.cache
def _sc_segsum():
    return pl.kernel(
        _sc_segsum_body,
        out_type=jax.ShapeDtypeStruct((NC, N_ACC, D), jnp.float32),
        mesh=_sc_mesh(),
        scratch_types=[
            pltpu.VMEM((CB, BK), jnp.int32),
            pltpu.VMEM((CB, BK), jnp.int32),
            pltpu.VMEM((BK, D), jnp.float32),
            pltpu.VMEM((BK, D), jnp.float32),
            pltpu.VMEM_SHARED((N_ACC, D), jnp.float32),
            pltpu.SemaphoreType.DMA,
            pltpu.SemaphoreType.DMA,
        ])


@functools.cache
def _sc_count():
    return pl.kernel(
        _sc_count_body,
        out_type=jax.ShapeDtypeStruct((NC, N_ACC, D), jnp.float32),
        mesh=_sc_mesh(),
        scratch_types=[
            pltpu.VMEM((CB, BK), jnp.int32),
            pltpu.VMEM((BK, D), jnp.float32),
            pltpu.VMEM((BK, D), jnp.float32),
            pltpu.VMEM_SHARED((N_ACC, D), jnp.float32),
        ])


def _dot_t(a, w):
    return lax.dot_general(a, w, (((1,), (1,)), ((), ())),
                           preferred_element_type=jnp.float32)


def _combine_mean(s_ref, cnt_ref):
    s = s_ref[0, :N, :] + s_ref[1, :N, :]
    cnt = cnt_ref[0, :N, 0:1] + cnt_ref[1, :N, 0:1]
    return s / jnp.maximum(cnt, 1.0)


def _bn_relu(z, g, b):
    mu = jnp.mean(z, axis=0, keepdims=True)
    var = jnp.mean((z - mu) ** 2, axis=0, keepdims=True)
    return jnp.maximum((z - mu) / jnp.sqrt(var + 1e-5) * g + b, 0.0)


def _tc1_body(s_ref, cnt_ref, x_ref, Wl_ref, bl_ref, Wr_ref, g_ref, b_ref,
              Wres1_ref, bres1_ref, Wres2_ref, bres2_ref, h1_ref, res2_ref):
    x = x_ref[...]
    mean = _combine_mean(s_ref, cnt_ref)
    z = _dot_t(mean, Wl_ref[...]) + bl_ref[...] + _dot_t(x, Wr_ref[...])
    h = _bn_relu(z, g_ref[...], b_ref[...])
    h = h + _dot_t(x, Wres1_ref[...]) + bres1_ref[...]
    h1_ref[...] = h
    res2_ref[...] = _dot_t(h, Wres2_ref[...]) + bres2_ref[...]


def _tc2_body(s_ref, cnt_ref, h_ref, res_ref, Wl_ref, bl_ref, Wr_ref,
              g_ref, b_ref, h2_ref):
    mean = _combine_mean(s_ref, cnt_ref)
    z = _dot_t(mean, Wl_ref[...]) + bl_ref[...] + _dot_t(h_ref[...], Wr_ref[...])
    h2_ref[...] = _bn_relu(z, g_ref[...], b_ref[...]) + res_ref[...]


def _tc3_body(s_ref, cnt_ref, h_ref, Wl_ref, bl_ref, Wr_ref,
              Wff1_ref, bff1_ref, Wff2_ref, bff2_ref,
              Wcls_ref, bcls_ref, out_ref):
    mean = _combine_mean(s_ref, cnt_ref)
    h3 = _dot_t(mean, Wl_ref[...]) + bl_ref[...] + _dot_t(h_ref[...], Wr_ref[...])
    f = jnp.maximum(_dot_t(h3, Wff1_ref[...]) + bff1_ref[...], 0.0)
    f = _dot_t(f, Wff2_ref[...]) + bff2_ref[...]
    out_ref[...] = _dot_t(f, Wcls_ref[...]) + bcls_ref[...]


_tc1 = pl.pallas_call(
    _tc1_body,
    out_shape=(jax.ShapeDtypeStruct((N, D), jnp.float32),
               jax.ShapeDtypeStruct((N, D), jnp.float32)))

_tc2 = pl.pallas_call(
    _tc2_body,
    out_shape=jax.ShapeDtypeStruct((N, D), jnp.float32))

_tc3 = pl.pallas_call(
    _tc3_body,
    out_shape=jax.ShapeDtypeStruct((N, 64), jnp.float32))


def kernel(x, edge_index, Wl1, bl1, Wr1, Wl2, bl2, Wr2, Wl3, bl3, Wr3,
           g1, b1, g2, b2, Wres1, bres1, Wres2, bres2,
           Wff1, bff1, Wff2, bff2, Wcls, bcls):
    src = edge_index[0].astype(jnp.int32)
    dst = edge_index[1].astype(jnp.int32)
    e = src.shape[0]
    pad = E_PAD - e
    srcs = jnp.concatenate([src, jnp.zeros((pad,), jnp.int32)]).reshape(TOTAL_B, BK)
    dsts = jnp.concatenate([dst, jnp.full((pad,), N, jnp.int32)]).reshape(TOTAL_B, BK)
    zd = jnp.zeros((BK, D), jnp.float32)
    ones = jnp.ones((BK, D), jnp.float32)

    r = lambda v: v.reshape(1, -1)

    cntp = _sc_count()(dsts, zd, ones)
    s1p = _sc_segsum()(x, srcs, dsts, zd)
    h1, res2 = _tc1(s1p, cntp, x, Wl1, r(bl1), Wr1, r(g1), r(b1),
                    Wres1, r(bres1), Wres2, r(bres2))
    s2p = _sc_segsum()(h1, srcs, dsts, zd)
    h2 = _tc2(s2p, cntp, h1, res2, Wl2, r(bl2), Wr2, r(g2), r(b2))
    s3p = _sc_segsum()(h2, srcs, dsts, zd)
    out = _tc3(s3p, cntp, h2, Wl3, r(bl3), Wr3,
               Wff1, r(bff1), Wff2, r(bff2), Wcls, r(bcls))
    return out


# speedup vs baseline: 1.0230x; 1.0230x over previous
import functools

import jax
import jax.numpy as jnp
from jax import lax
from jax.experimental import pallas as pl
from jax.experimental.pallas import tpu as pltpu
from jax.experimental.pallas import tpu_sc as plsc

N = 10000
D = 128
NC = 2
NS = 16
NW = NC * NS
EPW = 10240
BK = 64
K = EPW // BK
E_PAD = NW * EPW
TOTAL_B = E_PAD // BK
CB = 8
K0 = 288
K1 = 2 * K - K0
RPT = 640
N_ACC = NS * RPT


def _sc_segsum_body(h_hbm, srcs_hbm, dsts_hbm, zd_hbm, s_out,
                    src0_v, dst0_v, src1_v, dst1_v,
                    rows0_v, rows1_v, rows2_v, acc_sh,
                    gsem0, gsem1, gsem2, isem0, isem1):
    c = lax.axis_index("c")
    s = lax.axis_index("s")
    r0 = s * RPT
    bufs = (rows0_v, rows1_v, rows2_v)
    gsems = (gsem0, gsem1, gsem2)
    idx_sets = ((src0_v, dst0_v, isem0), (src1_v, dst1_v, isem1))
    pltpu.sync_copy(zd_hbm, rows0_v)
    for b in range(RPT // BK):
        pltpu.sync_copy(rows0_v, acc_sh.at[pl.ds(r0 + b * BK, BK)])
    plsc.subcore_barrier()

    base = jnp.where(c == 0, s * K0, NS * K0 + s * K1)
    nchunks = jnp.where(c == 0, K0 // CB, K1 // CB)

    def stage(ci, sv, dv, sem):
        b0 = base + ci * CB
        pltpu.async_copy(srcs_hbm.at[pl.ds(b0, CB)], sv, sem)
        pltpu.async_copy(dsts_hbm.at[pl.ds(b0, CB)], dv, sem)

    def wait_stage(ci, sv, dv, sem):
        b0 = base + ci * CB
        pltpu.make_async_copy(srcs_hbm.at[pl.ds(b0, CB)], sv, sem).wait()
        pltpu.make_async_copy(dsts_hbm.at[pl.ds(b0, CB)], dv, sem).wait()

    stage(0, *idx_sets[0])
    stage(1, *idx_sets[1])

    def trip(t, carry):
        for hh in range(2):
            ci = 2 * t + hh
            sv, dv, sem = idx_sets[hh]
            wait_stage(ci, sv, dv, sem)
            gd = [None, None, None]
            gd[0] = pltpu.async_copy(h_hbm.at[sv.at[0]], bufs[0], gsems[0])
            gd[1] = pltpu.async_copy(h_hbm.at[sv.at[1]], bufs[1], gsems[1])
            for j in range(CB):
                b = j % 3
                if j + 2 < CB:
                    b2 = (j + 2) % 3
                    gd[b2] = pltpu.async_copy(
                        h_hbm.at[sv.at[j + 2]], bufs[b2], gsems[b2])
                gd[b].wait()
                pltpu.sync_copy(bufs[b], acc_sh.at[dv.at[j]], add=True)

            @pl.when(ci + 2 < nchunks)
            def _():
                stage(ci + 2, sv, dv, sem)
        return carry

    lax.fori_loop(0, nchunks // 2, trip, 0)
    plsc.subcore_barrier()
    for b in range(RPT // BK):
        pltpu.sync_copy(acc_sh.at[pl.ds(r0 + b * BK, BK)], rows0_v)
        pltpu.sync_copy(rows0_v, s_out.at[c, pl.ds(r0 + b * BK, BK)])


def _sc_count_body(dsts_hbm, zd_hbm, ones_hbm,
                   c_out, dst_v, rows_v, ones_v, acc_sh):
    c = lax.axis_index("c")
    s = lax.axis_index("s")
    wid = c * NS + s
    r0 = s * RPT
    pltpu.sync_copy(zd_hbm, rows_v)
    for b in range(RPT // BK):
        pltpu.sync_copy(rows_v, acc_sh.at[pl.ds(r0 + b * BK, BK)])
    pltpu.sync_copy(ones_hbm, ones_v)
    plsc.subcore_barrier()

    def outer(t, carry):
        pltpu.sync_copy(dsts_hbm.at[pl.ds(wid * K + t * CB, CB)], dst_v)

        def step(j, carry2):
            pltpu.sync_copy(ones_v, acc_sh.at[dst_v.at[j]], add=True)
            return carry2

        lax.fori_loop(0, CB, step, 0)
        return carry

    lax.fori_loop(0, K // CB, outer, 0)
    plsc.subcore_barrier()
    for b in range(RPT // BK):
        pltpu.sync_copy(acc_sh.at[pl.ds(r0 + b * BK, BK)], rows_v)
        pltpu.sync_copy(rows_v, c_out.at[c, pl.ds(r0 + b * BK, BK)])


def _sc_mesh():
    return plsc.VectorSubcoreMesh(core_axis_name="c", subcore_axis_name="s")


@functools.cache
def _sc_segsum():
    return pl.kernel(
        _sc_segsum_body,
        out_type=jax.ShapeDtypeStruct((NC, N_ACC, D), jnp.float32),
        mesh=_sc_mesh(),
        scratch_types=[
            pltpu.VMEM((CB, BK), jnp.int32),
            pltpu.VMEM((CB, BK), jnp.int32),
            pltpu.VMEM((CB, BK), jnp.int32),
            pltpu.VMEM((CB, BK), jnp.int32),
            pltpu.VMEM((BK, D), jnp.float32),
            pltpu.VMEM((BK, D), jnp.float32),
            pltpu.VMEM((BK, D), jnp.float32),
            pltpu.VMEM_SHARED((N_ACC, D), jnp.float32),
            pltpu.SemaphoreType.DMA,
            pltpu.SemaphoreType.DMA,
            pltpu.SemaphoreType.DMA,
            pltpu.SemaphoreType.DMA,
            pltpu.SemaphoreType.DMA,
        ])


@functools.cache
def _sc_count():
    return pl.kernel(
        _sc_count_body,
        out_type=jax.ShapeDtypeStruct((NC, N_ACC, D), jnp.float32),
        mesh=_sc_mesh(),
        scratch_types=[
            pltpu.VMEM((CB, BK), jnp.int32),
            pltpu.VMEM((BK, D), jnp.float32),
            pltpu.VMEM((BK, D), jnp.float32),
            pltpu.VMEM_SHARED((N_ACC, D), jnp.float32),
        ])


def _dot_t(a, w):
    return lax.dot_general(a, w, (((1,), (1,)), ((), ())),
                           preferred_element_type=jnp.float32)


def _combine_mean(s_ref, cnt_ref):
    s = s_ref[0, :N, :] + s_ref[1, :N, :]
    cnt = cnt_ref[0, :N, 0:1] + cnt_ref[1, :N, 0:1]
    return s / jnp.maximum(cnt, 1.0)


def _bn_relu(z, g, b):
    mu = jnp.mean(z, axis=0, keepdims=True)
    var = jnp.mean((z - mu) ** 2, axis=0, keepdims=True)
    return jnp.maximum((z - mu) / jnp.sqrt(var + 1e-5) * g + b, 0.0)


def _tc1_body(s_ref, cnt_ref, x_ref, Wl_ref, bl_ref, Wr_ref, g_ref, b_ref,
              Wres1_ref, bres1_ref, Wres2_ref, bres2_ref, h1_ref, res2_ref):
    x = x_ref[...]
    mean = _combine_mean(s_ref, cnt_ref)
    z = _dot_t(mean, Wl_ref[...]) + bl_ref[...] + _dot_t(x, Wr_ref[...])
    h = _bn_relu(z, g_ref[...], b_ref[...])
    h = h + _dot_t(x, Wres1_ref[...]) + bres1_ref[...]
    h1_ref[...] = h
    res2_ref[...] = _dot_t(h, Wres2_ref[...]) + bres2_ref[...]


def _tc2_body(s_ref, cnt_ref, h_ref, res_ref, Wl_ref, bl_ref, Wr_ref,
              g_ref, b_ref, h2_ref):
    mean = _combine_mean(s_ref, cnt_ref)
    z = _dot_t(mean, Wl_ref[...]) + bl_ref[...] + _dot_t(h_ref[...], Wr_ref[...])
    h2_ref[...] = _bn_relu(z, g_ref[...], b_ref[...]) + res_ref[...]


def _tc3_body(s_ref, cnt_ref, h_ref, Wl_ref, bl_ref, Wr_ref,
              Wff1_ref, bff1_ref, Wff2_ref, bff2_ref,
              Wcls_ref, bcls_ref, out_ref):
    mean = _combine_mean(s_ref, cnt_ref)
    h3 = _dot_t(mean, Wl_ref[...]) + bl_ref[...] + _dot_t(h_ref[...], Wr_ref[...])
    f = jnp.maximum(_dot_t(h3, Wff1_ref[...]) + bff1_ref[...], 0.0)
    f = _dot_t(f, Wff2_ref[...]) + bff2_ref[...]
    out_ref[...] = _dot_t(f, Wcls_ref[...]) + bcls_ref[...]


_tc1 = pl.pallas_call(
    _tc1_body,
    out_shape=(jax.ShapeDtypeStruct((N, D), jnp.float32),
               jax.ShapeDtypeStruct((N, D), jnp.float32)))

_tc2 = pl.pallas_call(
    _tc2_body,
    out_shape=jax.ShapeDtypeStruct((N, D), jnp.float32))

_tc3 = pl.pallas_call(
    _tc3_body,
    out_shape=jax.ShapeDtypeStruct((N, 64), jnp.float32))


def kernel(x, edge_index, Wl1, bl1, Wr1, Wl2, bl2, Wr2, Wl3, bl3, Wr3,
           g1, b1, g2, b2, Wres1, bres1, Wres2, bres2,
           Wff1, bff1, Wff2, bff2, Wcls, bcls):
    src = edge_index[0].astype(jnp.int32)
    dst = edge_index[1].astype(jnp.int32)
    e = src.shape[0]
    pad = E_PAD - e
    srcs = jnp.concatenate([src, jnp.zeros((pad,), jnp.int32)]).reshape(TOTAL_B, BK)
    dsts = jnp.concatenate([dst, jnp.full((pad,), N, jnp.int32)]).reshape(TOTAL_B, BK)
    zd = jnp.zeros((BK, D), jnp.float32)
    ones = jnp.ones((BK, D), jnp.float32)

    r = lambda v: v.reshape(1, -1)

    cntp = _sc_count()(dsts, zd, ones)
    s1p = _sc_segsum()(x, srcs, dsts, zd)
    h1, res2 = _tc1(s1p, cntp, x, Wl1, r(bl1), Wr1, r(g1), r(b1),
                    Wres1, r(bres1), Wres2, r(bres2))
    s2p = _sc_segsum()(h1, srcs, dsts, zd)
    h2 = _tc2(s2p, cntp, h1, res2, Wl2, r(bl2), Wr2, r(g2), r(b2))
    s3p = _sc_segsum()(h2, srcs, dsts, zd)
    out = _tc3(s3p, cntp, h2, Wl3, r(bl3), Wr3,
               Wff1, r(bff1), Wff2, r(bff2), Wcls, r(bcls))
    return out
